# initial kernel scaffold (unmeasured)
import jax
import jax.numpy as jnp
from jax import lax
from jax.experimental import pallas as pl
from jax.experimental.pallas import tpu as pltpu


def kernel(x, dy):
    k, m = x.shape
    _, n = dy.shape
    m_half = m // 2

    def body(x_ref, dy_ref, out_ref, send_buf, recv_buf, send_sem, recv_sem):
        my_x = lax.axis_index("x")
        my_y = lax.axis_index("y")
        my_z = lax.axis_index("z")

        xb = x_ref[:].astype(jnp.bfloat16)
        dyb = dy_ref[:].astype(jnp.bfloat16)

        their_rows = (1 - my_x) * m_half
        a_theirs = lax.dynamic_slice(xb, (0, their_rows), (k, m_half))
        p_theirs = lax.dot_general(
            a_theirs, dyb, (((0,), (0,)), ((), ())),
            preferred_element_type=jnp.float32,
        )
        send_buf[:] = p_theirs.astype(jnp.bfloat16)

        rdma = pltpu.make_async_remote_copy(
            src_ref=send_buf,
            dst_ref=recv_buf,
            send_sem=send_sem,
            recv_sem=recv_sem,
            device_id=(1 - my_x, my_y, my_z),
            device_id_type=pl.DeviceIdType.MESH,
        )
        rdma.start()

        my_rows = my_x * m_half
        a_mine = lax.dynamic_slice(xb, (0, my_rows), (k, m_half))
        p_mine = lax.dot_general(
            a_mine, dyb, (((0,), (0,)), ((), ())),
            preferred_element_type=jnp.float32,
        )

        rdma.wait()
        out_ref[:] = p_mine + recv_buf[:].astype(jnp.float32)

    return pl.pallas_call(
        body,
        out_shape=jax.ShapeDtypeStruct((m_half, n), jnp.float32),
        in_specs=[
            pl.BlockSpec(memory_space=pltpu.VMEM),
            pl.BlockSpec(memory_space=pltpu.VMEM),
        ],
        out_specs=pl.BlockSpec(memory_space=pltpu.VMEM),
        scratch_shapes=[
            pltpu.VMEM((m_half, n), jnp.bfloat16),
            pltpu.VMEM((m_half, n), jnp.bfloat16),
            pltpu.SemaphoreType.DMA,
            pltpu.SemaphoreType.DMA,
        ],
        compiler_params=pltpu.CompilerParams(collective_id=0),
    )(x, dy)


# baseline (device time: 81483 ns/iter reference)
import jax
import jax.numpy as jnp
from jax import lax
from jax.experimental import pallas as pl
from jax.experimental.pallas import tpu as pltpu


def kernel(x, dy):
    k, m = x.shape
    _, n = dy.shape
    m_half = m // 2

    def body(x_ref, dy_ref, out_ref, send_buf, recv_buf, send_sem, recv_sem):
        my_x = lax.axis_index("x")
        my_y = lax.axis_index("y")
        my_z = lax.axis_index("z")

        dyb = dy_ref[:].astype(jnp.bfloat16)

        their_rows = (1 - my_x) * m_half
        a_theirs = x_ref[:, pl.ds(their_rows, m_half)].astype(jnp.bfloat16)
        p_theirs = lax.dot_general(
            a_theirs, dyb, (((0,), (0,)), ((), ())),
            preferred_element_type=jnp.float32,
        )
        send_buf[:] = p_theirs.astype(jnp.bfloat16)

        rdma = pltpu.make_async_remote_copy(
            src_ref=send_buf,
            dst_ref=recv_buf,
            send_sem=send_sem,
            recv_sem=recv_sem,
            device_id=(1 - my_x, my_y, my_z),
            device_id_type=pl.DeviceIdType.MESH,
        )
        rdma.start()

        my_rows = my_x * m_half
        a_mine = x_ref[:, pl.ds(my_rows, m_half)].astype(jnp.bfloat16)
        p_mine = lax.dot_general(
            a_mine, dyb, (((0,), (0,)), ((), ())),
            preferred_element_type=jnp.float32,
        )

        rdma.wait()
        out_ref[:] = p_mine + recv_buf[:].astype(jnp.float32)

    return pl.pallas_call(
        body,
        out_shape=jax.ShapeDtypeStruct((m_half, n), jnp.float32),
        in_specs=[
            pl.BlockSpec(memory_space=pltpu.VMEM),
            pl.BlockSpec(memory_space=pltpu.VMEM),
        ],
        out_specs=pl.BlockSpec(memory_space=pltpu.VMEM),
        scratch_shapes=[
            pltpu.VMEM((m_half, n), jnp.bfloat16),
            pltpu.VMEM((m_half, n), jnp.bfloat16),
            pltpu.SemaphoreType.DMA,
            pltpu.SemaphoreType.DMA,
        ],
        compiler_params=pltpu.CompilerParams(
            vmem_limit_bytes=100 * 1024 * 1024,
        ),
    )(x, dy)


# device time: 65296 ns/iter; 1.2479x vs baseline; 1.2479x over previous
import jax
import jax.numpy as jnp
from jax import lax
from jax.experimental import pallas as pl
from jax.experimental.pallas import tpu as pltpu

N_RING = 8
N_FWD = 4
N_BWD = 3


def kernel(x, dy):
    k, m = x.shape
    _, n = dy.shape
    m_half = m // 2
    blk = n // N_RING

    def body(x_ref, dy_ref, out_ref, blks, xsend, xrecv,
             fwd_send, fwd_recv, bwd_send, bwd_recv, xs_sem, xr_sem):
        my_x = lax.axis_index("x")
        my_y = lax.axis_index("y")
        my_z = lax.axis_index("z")

        p = jnp.where(my_y == 0, my_z, 7 - my_z)
        fwd_y = jnp.where(my_y == 0,
                          jnp.where(my_z == 3, 1, 0),
                          jnp.where(my_z == 0, 0, 1))
        fwd_z = jnp.where(my_y == 0,
                          jnp.where(my_z == 3, 3, my_z + 1),
                          jnp.where(my_z == 0, 0, my_z - 1))
        bwd_y = jnp.where(my_y == 0,
                          jnp.where(my_z == 0, 1, 0),
                          jnp.where(my_z == 3, 0, 1))
        bwd_z = jnp.where(my_y == 0,
                          jnp.where(my_z == 0, 0, my_z - 1),
                          jnp.where(my_z == 3, 3, my_z + 1))

        dyb = dy_ref[:, pl.ds(p * blk, blk)].astype(jnp.bfloat16)

        their_rows = (1 - my_x) * m_half
        a_theirs = x_ref[:, pl.ds(their_rows, m_half)].astype(jnp.bfloat16)
        xsend[:] = lax.dot_general(
            a_theirs, dyb, (((0,), (0,)), ((), ())),
            preferred_element_type=jnp.float32,
        ).astype(jnp.bfloat16)

        x_rdma = pltpu.make_async_remote_copy(
            src_ref=xsend, dst_ref=xrecv,
            send_sem=xs_sem, recv_sem=xr_sem,
            device_id=(1 - my_x, my_y, my_z),
            device_id_type=pl.DeviceIdType.MESH,
        )
        x_rdma.start()

        a_mine = x_ref[:, pl.ds(my_x * m_half, m_half)].astype(jnp.bfloat16)
        p_mine = lax.dot_general(
            a_mine, dyb, (((0,), (0,)), ((), ())),
            preferred_element_type=jnp.float32,
        )

        x_rdma.wait()
        blks[pl.ds(p, 1)] = (
            p_mine + xrecv[:].astype(jnp.float32)
        ).astype(jnp.bfloat16)[None]

        def fwd_desc(hop):
            b = (p - hop) % N_RING
            return pltpu.make_async_remote_copy(
                src_ref=blks.at[b], dst_ref=blks.at[b],
                send_sem=fwd_send.at[hop], recv_sem=fwd_recv.at[hop],
                device_id=(my_x, fwd_y, fwd_z),
                device_id_type=pl.DeviceIdType.MESH,
            )

        def bwd_desc(hop):
            b = (p + hop) % N_RING
            return pltpu.make_async_remote_copy(
                src_ref=blks.at[b], dst_ref=blks.at[b],
                send_sem=bwd_send.at[hop], recv_sem=bwd_recv.at[hop],
                device_id=(my_x, bwd_y, bwd_z),
                device_id_type=pl.DeviceIdType.MESH,
            )

        fwd = [fwd_desc(h) for h in range(N_FWD)]
        bwd = [bwd_desc(h) for h in range(N_BWD)]

        fwd[0].start()
        bwd[0].start()
        for h in range(1, N_FWD):
            fwd[h - 1].wait_recv()
            fwd[h].start()
            if h < N_BWD:
                bwd[h - 1].wait_recv()
                bwd[h].start()
        fwd[N_FWD - 1].wait_recv()
        bwd[N_BWD - 1].wait_recv()

        for q in range(N_RING):
            out_ref[:, q * blk:(q + 1) * blk] = blks[q].astype(jnp.float32)

        for d in fwd + bwd:
            d.wait_send()

    return pl.pallas_call(
        body,
        out_shape=jax.ShapeDtypeStruct((m_half, n), jnp.float32),
        in_specs=[
            pl.BlockSpec(memory_space=pltpu.VMEM),
            pl.BlockSpec(memory_space=pltpu.VMEM),
        ],
        out_specs=pl.BlockSpec(memory_space=pltpu.VMEM),
        scratch_shapes=[
            pltpu.VMEM((N_RING, m_half, blk), jnp.bfloat16),
            pltpu.VMEM((m_half, blk), jnp.bfloat16),
            pltpu.VMEM((m_half, blk), jnp.bfloat16),
            pltpu.SemaphoreType.DMA((N_FWD,)),
            pltpu.SemaphoreType.DMA((N_FWD,)),
            pltpu.SemaphoreType.DMA((N_BWD,)),
            pltpu.SemaphoreType.DMA((N_BWD,)),
            pltpu.SemaphoreType.DMA,
            pltpu.SemaphoreType.DMA,
        ],
        compiler_params=pltpu.CompilerParams(
            vmem_limit_bytes=100 * 1024 * 1024,
        ),
    )(x, dy)
